# baseline (speedup 1.0000x reference)
import functools

import jax
import jax.numpy as jnp
from jax import lax
from jax.experimental import pallas as pl
from jax.experimental.pallas import tpu as pltpu
from jax.experimental.pallas import tpu_sc as plsc

B = 16384
EMB = 16
POI_IN = 85
POI_OUT = 10
OUT_W = 4 * EMB + POI_OUT
N_DAYS = 75
N_SLOTS = 48
N_CITIES = 200

_INFO = plsc.get_sparse_core_info()
_NC = _INFO.num_cores
_NS = _INFO.num_subcores
_NW = _NC * _NS
_BPW = B // _NW
_L = 16
_G = _BPW // _L
_NBUF = 2 * _L


def _poi_body(w_ref, x_ref, b_ref, o_ref):
    acc = jnp.dot(w_ref[...], x_ref[...], preferred_element_type=jnp.float32)
    o_ref[pl.ds(0, POI_OUT), :] = jnp.maximum(acc + b_ref[...], 0.0)
    o_ref[pl.ds(POI_OUT, 16 - POI_OUT), :] = jnp.zeros(
        (16 - POI_OUT, B), jnp.float32)


def _poi_matmul(w, x_t, b2):
    return pl.pallas_call(
        _poi_body,
        out_shape=jax.ShapeDtypeStruct((16, B), jnp.float32),
    )(w, x_t, b2)


def _stripe_body(sc_ref, epoi_ref, o_ref):
    o_ref[...] = epoi_ref[...]


def _stripe_merge(sc_out, epoi):
    blk = 4096
    return pl.pallas_call(
        _stripe_body,
        grid=(B // blk,),
        in_specs=[
            pl.BlockSpec(memory_space=pl.ANY),
            pl.BlockSpec((16, blk), lambda i: (0, i)),
        ],
        out_specs=pl.BlockSpec((16, blk), lambda i: (4, i)),
        out_shape=jax.ShapeDtypeStruct((80, B), jnp.float32),
        input_output_aliases={0: 0},
    )(sc_out, epoi)


def _sc_body(uidh, dh, th, cityh, uid_t, day_t, slot_t, city_t,
             out_hbm, uid_v, d_v, t_v, c_v, minib, out_v, day_v, slot_v, city_v,
             sem):
    wid = lax.axis_index("s") * _NC + lax.axis_index("c")
    base = wid * _BPW

    pltpu.sync_copy(uidh.at[pl.ds(base, _BPW)], uid_v)
    pltpu.sync_copy(dh.at[pl.ds(base, _BPW)], d_v)
    pltpu.sync_copy(th.at[pl.ds(base, _BPW)], t_v)
    pltpu.sync_copy(cityh.at[pl.ds(base, _BPW)], c_v)
    pltpu.sync_copy(day_t, day_v)
    pltpu.sync_copy(slot_t, slot_v)
    pltpu.sync_copy(city_t, city_v)

    lane = lax.iota(jnp.int32, _L)
    svecs = [jnp.full((_L,), s, dtype=jnp.int32) for s in range(EMB)]
    small = ((0, day_v, d_v), (1, slot_v, t_v), (2, city_v, c_v))

    def fire(g):
        vec = uid_v[pl.ds(g * _L, _L)]
        bank = lax.rem(g, 2) * _L
        for r in range(_L):
            u = jnp.sum(jnp.where(lane == r, vec, 0))
            q = pl.multiple_of((u // 128) * 128, 128)
            pltpu.async_copy(uid_t.at[:, pl.ds(q, 128)], minib.at[bank + r],
                             sem)

    def smalls(c):
        ocols = lane + c * _L
        for k, tbl, iv in small:
            col = iv[pl.ds(c * _L, _L)]
            for s in range(EMB):
                vals = plsc.load_gather(tbl, [svecs[s], col])
                orows = jnp.full((_L,), (k + 1) * EMB + s, dtype=jnp.int32)
                plsc.store_scatter(out_v, [orows, ocols], vals)

    def drain(g):
        vec = uid_v[pl.ds(g * _L, _L)]
        bank = lax.rem(g, 2) * _L
        for r in range(_L):
            pltpu.make_async_copy(uid_t.at[:, pl.ds(0, 128)],
                                  minib.at[bank + r], sem).wait()
        for r in range(_L):
            u = jnp.sum(jnp.where(lane == r, vec, 0))
            off = jnp.full((_L,), lax.rem(u, 128), dtype=jnp.int32)
            bvec = jnp.full((_L,), bank + r, dtype=jnp.int32)
            vals = plsc.load_gather(minib, [bvec, lane, off])
            cols = jnp.full((_L,), g * _L + r, dtype=jnp.int32)
            plsc.store_scatter(out_v, [lane, cols], vals)

    fire(0)

    @pl.loop(1, _G)
    def _(g):
        fire(g)
        smalls(g - 1)
        drain(g - 1)

    smalls(_G - 1)
    drain(_G - 1)

    pltpu.sync_copy(out_v, out_hbm.at[pl.ds(0, 4 * EMB), pl.ds(base, _BPW)])


_sc_fused = functools.partial(
    pl.kernel,
    out_type=jax.ShapeDtypeStruct((80, B), jnp.float32),
    mesh=plsc.VectorSubcoreMesh(core_axis_name="c", subcore_axis_name="s"),
    scratch_types=[
        pltpu.VMEM((_BPW,), jnp.int32),
        pltpu.VMEM((_BPW,), jnp.int32),
        pltpu.VMEM((_BPW,), jnp.int32),
        pltpu.VMEM((_BPW,), jnp.int32),
        pltpu.VMEM((_NBUF, EMB, 128), jnp.float32),
        pltpu.VMEM((4 * EMB, _BPW), jnp.float32),
        pltpu.VMEM((EMB, N_DAYS), jnp.float32),
        pltpu.VMEM((EMB, N_SLOTS), jnp.float32),
        pltpu.VMEM((EMB, N_CITIES), jnp.float32),
        pltpu.SemaphoreType.DMA,
    ],
    compiler_params=pltpu.CompilerParams(needs_layout_passes=False,
                                         disable_bounds_checks=True),
)(_sc_body)


@jax.jit
def kernel(uid, d, t, city, poi, uid_table, day_table, slot_table,
           city_table, poi_W, poi_b):
    epoi_t = _poi_matmul(poi_W, poi.T, poi_b.reshape(POI_OUT, 1))
    sc_out = _sc_fused(uid.astype(jnp.int32), d.astype(jnp.int32),
                       t.astype(jnp.int32), city.astype(jnp.int32),
                       uid_table.T, day_table.T, slot_table.T, city_table.T)
    out_t = _stripe_merge(sc_out, epoi_t)
    return out_t[:OUT_W].T


# speedup vs baseline: 1.0875x; 1.0875x over previous
import functools

import jax
import jax.numpy as jnp
from jax import lax
from jax.experimental import pallas as pl
from jax.experimental.pallas import tpu as pltpu
from jax.experimental.pallas import tpu_sc as plsc

B = 16384
EMB = 16
POI_IN = 85
POI_OUT = 10
OUT_W = 4 * EMB + POI_OUT
N_DAYS = 75
N_SLOTS = 48
N_CITIES = 200

_INFO = plsc.get_sparse_core_info()
_NC = _INFO.num_cores
_NS = _INFO.num_subcores
_NW = _NC * _NS
_BPW = B // _NW
_L = 16
_G = _BPW // _L
_NBUF = 2 * _L


def _poi_body(w_ref, x_ref, b_ref, o_ref):
    acc = jnp.dot(w_ref[...], x_ref[...], preferred_element_type=jnp.float32)
    o_ref[...] = jnp.maximum(acc + b_ref[...], 0.0)


def _poi_matmul(w, x_t, b2):
    return pl.pallas_call(
        _poi_body,
        out_shape=jax.ShapeDtypeStruct((POI_OUT, B), jnp.float32),
    )(w, x_t, b2)


def _sc_body(uidh, dh, th, cityh, uid_t, day_t, slot_t, city_t, epoi_hbm,
             out_hbm, uid_v, d_v, t_v, c_v, minib, out_v, day_v, slot_v,
             city_v, sem):
    wid = lax.axis_index("s") * _NC + lax.axis_index("c")
    base = wid * _BPW

    pltpu.sync_copy(uidh.at[pl.ds(base, _BPW)], uid_v)
    pltpu.sync_copy(dh.at[pl.ds(base, _BPW)], d_v)
    pltpu.sync_copy(th.at[pl.ds(base, _BPW)], t_v)
    pltpu.sync_copy(cityh.at[pl.ds(base, _BPW)], c_v)
    pltpu.sync_copy(day_t, day_v)
    pltpu.sync_copy(slot_t, slot_v)
    pltpu.sync_copy(city_t, city_v)

    lane = lax.iota(jnp.int32, _L)
    svecs = [jnp.full((_L,), s, dtype=jnp.int32) for s in range(EMB)]
    small = ((0, day_v, d_v), (1, slot_v, t_v), (2, city_v, c_v))

    def fire(g):
        vec = uid_v[pl.ds(g * _L, _L)]
        bank = lax.rem(g, 2) * _L
        for r in range(_L):
            u = jnp.sum(jnp.where(lane == r, vec, 0))
            q = pl.multiple_of((u // 128) * 128, 128)
            pltpu.async_copy(uid_t.at[:, pl.ds(q, 128)], minib.at[bank + r],
                             sem)

    def smalls(c):
        ocols = lane + c * _L
        for k, tbl, iv in small:
            col = iv[pl.ds(c * _L, _L)]
            for s in range(EMB):
                vals = plsc.load_gather(tbl, [svecs[s], col])
                orows = jnp.full((_L,), (k + 1) * EMB + s, dtype=jnp.int32)
                plsc.store_scatter(out_v, [orows, ocols], vals)

    def drain(g):
        vec = uid_v[pl.ds(g * _L, _L)]
        bank = lax.rem(g, 2) * _L
        for r in range(_L):
            pltpu.make_async_copy(uid_t.at[:, pl.ds(0, 128)],
                                  minib.at[bank + r], sem).wait()
        for r in range(_L):
            u = jnp.sum(jnp.where(lane == r, vec, 0))
            off = jnp.full((_L,), lax.rem(u, 128), dtype=jnp.int32)
            bvec = jnp.full((_L,), bank + r, dtype=jnp.int32)
            vals = plsc.load_gather(minib, [bvec, lane, off])
            cols = jnp.full((_L,), g * _L + r, dtype=jnp.int32)
            plsc.store_scatter(out_v, [lane, cols], vals)

    fire(0)

    @pl.loop(1, _G)
    def _(g):
        fire(g)
        smalls(g - 1)
        drain(g - 1)

    smalls(_G - 1)
    pltpu.sync_copy(epoi_hbm.at[:, pl.ds(base, _BPW)],
                    out_v.at[pl.ds(4 * EMB, POI_OUT), :])
    drain(_G - 1)

    pltpu.sync_copy(out_v, out_hbm.at[:, pl.ds(base, _BPW)])


_sc_fused = functools.partial(
    pl.kernel,
    out_type=jax.ShapeDtypeStruct((OUT_W, B), jnp.float32),
    mesh=plsc.VectorSubcoreMesh(core_axis_name="c", subcore_axis_name="s"),
    scratch_types=[
        pltpu.VMEM((_BPW,), jnp.int32),
        pltpu.VMEM((_BPW,), jnp.int32),
        pltpu.VMEM((_BPW,), jnp.int32),
        pltpu.VMEM((_BPW,), jnp.int32),
        pltpu.VMEM((_NBUF, EMB, 128), jnp.float32),
        pltpu.VMEM((OUT_W, _BPW), jnp.float32),
        pltpu.VMEM((EMB, N_DAYS), jnp.float32),
        pltpu.VMEM((EMB, N_SLOTS), jnp.float32),
        pltpu.VMEM((EMB, N_CITIES), jnp.float32),
        pltpu.SemaphoreType.DMA,
    ],
    compiler_params=pltpu.CompilerParams(needs_layout_passes=False,
                                         disable_bounds_checks=True),
)(_sc_body)


@jax.jit
def kernel(uid, d, t, city, poi, uid_table, day_table, slot_table,
           city_table, poi_W, poi_b):
    epoi_t = _poi_matmul(poi_W, poi.T, poi_b.reshape(POI_OUT, 1))
    out_t = _sc_fused(uid.astype(jnp.int32), d.astype(jnp.int32),
                      t.astype(jnp.int32), city.astype(jnp.int32),
                      uid_table.T, day_table.T, slot_table.T, city_table.T,
                      epoi_t)
    return out_t.T


# speedup vs baseline: 1.1309x; 1.0399x over previous
import functools

import jax
import jax.numpy as jnp
from jax import lax
from jax.experimental import pallas as pl
from jax.experimental.pallas import tpu as pltpu
from jax.experimental.pallas import tpu_sc as plsc

B = 16384
EMB = 16
POI_IN = 85
POI_OUT = 10
OUT_W = 4 * EMB + POI_OUT
N_DAYS = 75
N_SLOTS = 48
N_CITIES = 200

_INFO = plsc.get_sparse_core_info()
_NC = _INFO.num_cores
_NS = _INFO.num_subcores
_NW = _NC * _NS
_BPW = B // _NW
_L = 16
_G = _BPW // _L
_NBUF = 2 * _L


def _poi_body(w_ref, x_ref, b_ref, o_ref):
    acc = jnp.dot(w_ref[...], x_ref[...], preferred_element_type=jnp.float32)
    o_ref[...] = jnp.maximum(acc + b_ref[...], 0.0)


def _poi_matmul(w, x_t, b2):
    return pl.pallas_call(
        _poi_body,
        out_shape=jax.ShapeDtypeStruct((POI_OUT, B), jnp.float32),
    )(w, x_t, b2)


def _sc_body(uidh, dh, th, cityh, uid_t, day_t, slot_t, city_t, epoi_hbm,
             out_hbm, uid_v, d_v, t_v, c_v, minib, out_v, day_v, slot_v,
             city_v, sem, semp):
    wid = lax.axis_index("s") * _NC + lax.axis_index("c")
    base = wid * _BPW

    h_uid = pltpu.async_copy(uidh.at[pl.ds(base, _BPW)], uid_v, sem)
    pro = [
        pltpu.async_copy(dh.at[pl.ds(base, _BPW)], d_v, semp),
        pltpu.async_copy(th.at[pl.ds(base, _BPW)], t_v, semp),
        pltpu.async_copy(cityh.at[pl.ds(base, _BPW)], c_v, semp),
        pltpu.async_copy(day_t, day_v, semp),
        pltpu.async_copy(slot_t, slot_v, semp),
        pltpu.async_copy(city_t, city_v, semp),
    ]

    lane = lax.iota(jnp.int32, _L)
    svecs = [jnp.full((_L,), s, dtype=jnp.int32) for s in range(EMB)]
    small = ((0, day_v, d_v), (1, slot_v, t_v), (2, city_v, c_v))

    def fire(g):
        vec = uid_v[pl.ds(g * _L, _L)]
        bank = lax.rem(g, 2) * _L
        for r in range(_L):
            u = jnp.sum(jnp.where(lane == r, vec, 0))
            q = pl.multiple_of((u // 128) * 128, 128)
            pltpu.async_copy(uid_t.at[:, pl.ds(q, 128)], minib.at[bank + r],
                             sem)

    def smalls(c):
        ocols = lane + c * _L
        for k, tbl, iv in small:
            col = iv[pl.ds(c * _L, _L)]
            for s in range(EMB):
                vals = plsc.load_gather(tbl, [svecs[s], col])
                orows = jnp.full((_L,), (k + 1) * EMB + s, dtype=jnp.int32)
                plsc.store_scatter(out_v, [orows, ocols], vals)

    def drain(g):
        vec = uid_v[pl.ds(g * _L, _L)]
        bank = lax.rem(g, 2) * _L
        for r in range(_L):
            pltpu.make_async_copy(uid_t.at[:, pl.ds(0, 128)],
                                  minib.at[bank + r], sem).wait()
        for r in range(_L):
            u = jnp.sum(jnp.where(lane == r, vec, 0))
            off = jnp.full((_L,), lax.rem(u, 128), dtype=jnp.int32)
            bvec = jnp.full((_L,), bank + r, dtype=jnp.int32)
            vals = plsc.load_gather(minib, [bvec, lane, off])
            cols = jnp.full((_L,), g * _L + r, dtype=jnp.int32)
            plsc.store_scatter(out_v, [lane, cols], vals)

    h_uid.wait()
    fire(0)
    for h in pro:
        h.wait()

    @pl.loop(1, _G)
    def _(g):
        fire(g)
        smalls(g - 1)
        drain(g - 1)

    smalls(_G - 1)
    pltpu.sync_copy(epoi_hbm.at[:, pl.ds(base, _BPW)],
                    out_v.at[pl.ds(4 * EMB, POI_OUT), :])
    drain(_G - 1)

    pltpu.sync_copy(out_v, out_hbm.at[:, pl.ds(base, _BPW)])


_sc_fused = functools.partial(
    pl.kernel,
    out_type=jax.ShapeDtypeStruct((OUT_W, B), jnp.float32),
    mesh=plsc.VectorSubcoreMesh(core_axis_name="c", subcore_axis_name="s"),
    scratch_types=[
        pltpu.VMEM((_BPW,), jnp.int32),
        pltpu.VMEM((_BPW,), jnp.int32),
        pltpu.VMEM((_BPW,), jnp.int32),
        pltpu.VMEM((_BPW,), jnp.int32),
        pltpu.VMEM((_NBUF, EMB, 128), jnp.float32),
        pltpu.VMEM((OUT_W, _BPW), jnp.float32),
        pltpu.VMEM((EMB, N_DAYS), jnp.float32),
        pltpu.VMEM((EMB, N_SLOTS), jnp.float32),
        pltpu.VMEM((EMB, N_CITIES), jnp.float32),
        pltpu.SemaphoreType.DMA,
        pltpu.SemaphoreType.DMA,
    ],
    compiler_params=pltpu.CompilerParams(needs_layout_passes=False,
                                         disable_bounds_checks=True),
)(_sc_body)


@jax.jit
def kernel(uid, d, t, city, poi, uid_table, day_table, slot_table,
           city_table, poi_W, poi_b):
    epoi_t = _poi_matmul(poi_W, poi.T, poi_b.reshape(POI_OUT, 1))
    out_t = _sc_fused(uid.astype(jnp.int32), d.astype(jnp.int32),
                      t.astype(jnp.int32), city.astype(jnp.int32),
                      uid_table.T, day_table.T, slot_table.T, city_table.T,
                      epoi_t)
    return out_t.T


# speedup vs baseline: 1.1448x; 1.0123x over previous
import functools

import jax
import jax.numpy as jnp
from jax import lax
from jax.experimental import pallas as pl
from jax.experimental.pallas import tpu as pltpu
from jax.experimental.pallas import tpu_sc as plsc

B = 16384
EMB = 16
POI_IN = 85
POI_OUT = 10
OUT_W = 4 * EMB + POI_OUT
N_DAYS = 75
N_SLOTS = 48
N_CITIES = 200

_INFO = plsc.get_sparse_core_info()
_NC = _INFO.num_cores
_NS = _INFO.num_subcores
_NW = _NC * _NS
_BPW = B // _NW
_L = 16
_G = _BPW // _L
_NBUF = 2 * _L


def _poi_body(w_ref, x_ref, b_ref, o_ref):
    acc = jnp.dot(w_ref[...], x_ref[...], preferred_element_type=jnp.float32)
    o_ref[...] = jnp.maximum(acc + b_ref[...], 0.0)


def _poi_matmul(w, x_t, b2):
    return pl.pallas_call(
        _poi_body,
        out_shape=jax.ShapeDtypeStruct((POI_OUT, B), jnp.float32),
    )(w, x_t, b2)


def _sc_body(uidh, dh, th, cityh, uid_t, day_t, slot_t, city_t, epoi_hbm,
             out_hbm, uid_v, d_v, t_v, c_v, minib, out_v, day_v, slot_v,
             city_v, sem, semp):
    wid = lax.axis_index("s") * _NC + lax.axis_index("c")
    base = wid * _BPW

    h_uid = pltpu.async_copy(uidh.at[pl.ds(base, _BPW)], uid_v, sem)
    pro = [
        pltpu.async_copy(dh.at[pl.ds(base, _BPW)], d_v, semp),
        pltpu.async_copy(th.at[pl.ds(base, _BPW)], t_v, semp),
        pltpu.async_copy(cityh.at[pl.ds(base, _BPW)], c_v, semp),
        pltpu.async_copy(day_t, day_v, semp),
        pltpu.async_copy(slot_t, slot_v, semp),
        pltpu.async_copy(city_t, city_v, semp),
    ]

    lane = lax.iota(jnp.int32, _L)
    svecs = [jnp.full((_L,), s, dtype=jnp.int32) for s in range(EMB)]
    small = ((0, day_v, d_v), (1, slot_v, t_v), (2, city_v, c_v))

    def fire(g):
        vec = uid_v[pl.ds(g * _L, _L)]
        bank = lax.rem(g, 2) * _L
        for r in range(_L):
            u = jnp.sum(jnp.where(lane == r, vec, 0))
            q = pl.multiple_of((u // 128) * 128, 128)
            pltpu.async_copy(uid_t.at[:, pl.ds(q, 128)], minib.at[bank + r],
                             sem)

    def smalls(c):
        ocols = lane + c * _L
        for k, tbl, iv in small:
            col = iv[pl.ds(c * _L, _L)]
            for s in range(EMB):
                vals = plsc.load_gather(tbl, [svecs[s], col])
                orows = jnp.full((_L,), (k + 1) * EMB + s, dtype=jnp.int32)
                plsc.store_scatter(out_v, [orows, ocols], vals)

    def drain(g):
        vec = uid_v[pl.ds(g * _L, _L)]
        bank = lax.rem(g, 2) * _L
        for r in range(_L):
            pltpu.make_async_copy(uid_t.at[:, pl.ds(0, 128)],
                                  minib.at[bank + r], sem).wait()
        for r in range(_L):
            u = jnp.sum(jnp.where(lane == r, vec, 0))
            off = jnp.full((_L,), lax.rem(u, 128), dtype=jnp.int32)
            bvec = jnp.full((_L,), bank + r, dtype=jnp.int32)
            vals = plsc.load_gather(minib, [bvec, lane, off])
            cols = jnp.full((_L,), g * _L + r, dtype=jnp.int32)
            plsc.store_scatter(out_v, [lane, cols], vals)

    h_uid.wait()
    fire(0)
    for h in pro:
        h.wait()

    @pl.loop(1, _G)
    def _(g):
        fire(g)
        smalls(g - 1)
        drain(g - 1)

    h_epoi = pltpu.async_copy(epoi_hbm.at[:, pl.ds(base, _BPW)],
                              out_v.at[pl.ds(4 * EMB, POI_OUT), :], semp)
    smalls(_G - 1)
    drain(_G - 1)
    h_epoi.wait()

    pltpu.sync_copy(out_v, out_hbm.at[:, pl.ds(base, _BPW)])


_sc_fused = functools.partial(
    pl.kernel,
    out_type=jax.ShapeDtypeStruct((OUT_W, B), jnp.float32),
    mesh=plsc.VectorSubcoreMesh(core_axis_name="c", subcore_axis_name="s"),
    scratch_types=[
        pltpu.VMEM((_BPW,), jnp.int32),
        pltpu.VMEM((_BPW,), jnp.int32),
        pltpu.VMEM((_BPW,), jnp.int32),
        pltpu.VMEM((_BPW,), jnp.int32),
        pltpu.VMEM((_NBUF, EMB, 128), jnp.float32),
        pltpu.VMEM((OUT_W, _BPW), jnp.float32),
        pltpu.VMEM((EMB, N_DAYS), jnp.float32),
        pltpu.VMEM((EMB, N_SLOTS), jnp.float32),
        pltpu.VMEM((EMB, N_CITIES), jnp.float32),
        pltpu.SemaphoreType.DMA,
        pltpu.SemaphoreType.DMA,
    ],
    compiler_params=pltpu.CompilerParams(needs_layout_passes=False,
                                         disable_bounds_checks=True),
)(_sc_body)


@jax.jit
def kernel(uid, d, t, city, poi, uid_table, day_table, slot_table,
           city_table, poi_W, poi_b):
    epoi_t = _poi_matmul(poi_W, poi.T, poi_b.reshape(POI_OUT, 1))
    out_t = _sc_fused(uid.astype(jnp.int32), d.astype(jnp.int32),
                      t.astype(jnp.int32), city.astype(jnp.int32),
                      uid_table.T, day_table.T, slot_table.T, city_table.T,
                      epoi_t)
    return out_t.T
